# Initial kernel scaffold; baseline (speedup 1.0000x reference)
import functools

import jax
import jax.numpy as jnp
from jax import lax
from jax.experimental import pallas as pl
from jax.experimental.pallas import tpu as pltpu
from jax.experimental.pallas import tpu_sc as plsc

N = 10000
E = 320000
D = 128

NC = 2
NS = 16
NW = NC * NS
EPW = E // NW
CH = 80
NCHUNK = EPW // CH
TPN = N // NS
CW = 16


def _agg_kernel_body(x_hbm, srcg_hbm, dstg_hbm, zrow_hbm, zcnt_hbm, ones_hbm,
                     outp_hbm, outc_hbm,
                     src_v, dst_v, rows_v, ones_v, stage_v, cstage_v, sem):
    cid = lax.axis_index("c")
    sid = lax.axis_index("s")
    wid = cid * NS + sid
    base = sid * TPN

    pltpu.sync_copy(srcg_hbm.at[wid], src_v)
    pltpu.sync_copy(dstg_hbm.at[wid], dst_v)
    pltpu.sync_copy(ones_hbm, ones_v)
    pltpu.sync_copy(zrow_hbm, stage_v)
    pltpu.sync_copy(zcnt_hbm, cstage_v)

    def _inner(agg_sh, cnt_sh):
        for r in range(TPN // 125):
            pltpu.sync_copy(stage_v, agg_sh.at[pl.ds(base + r * 125, 125)])
        pltpu.sync_copy(cstage_v, cnt_sh.at[pl.ds(base, TPN)])
        plsc.subcore_barrier()

        @pl.loop(0, NCHUNK)
        def _(j):
            pltpu.async_copy(x_hbm.at[src_v.at[j]], rows_v, sem).wait()
            pltpu.sync_copy(rows_v, agg_sh.at[dst_v.at[j]], add=True)
            pltpu.sync_copy(ones_v, cnt_sh.at[dst_v.at[j]], add=True)

        plsc.subcore_barrier()

        for r in range(TPN // 125):
            pltpu.sync_copy(agg_sh.at[pl.ds(base + r * 125, 125)], stage_v)
            pltpu.sync_copy(stage_v, outp_hbm.at[cid, pl.ds(base + r * 125, 125)])
        pltpu.sync_copy(cnt_sh.at[pl.ds(base, TPN)], cstage_v)
        pltpu.sync_copy(cstage_v, outc_hbm.at[cid, pl.ds(base, TPN)])

    pl.run_scoped(
        _inner,
        pltpu.VMEM_SHARED((N, D), jnp.float32),
        pltpu.VMEM_SHARED((N, CW), jnp.float32),
    )


def _sc_aggregate(x, srcg, dstg, zrow, zcnt, ones):
    mesh = plsc.VectorSubcoreMesh(core_axis_name="c", subcore_axis_name="s")
    return pl.kernel(
        _agg_kernel_body,
        out_type=[
            jax.ShapeDtypeStruct((NC, N, D), jnp.float32),
            jax.ShapeDtypeStruct((NC, N, CW), jnp.float32),
        ],
        mesh=mesh,
        scratch_types=[
            pltpu.VMEM((NCHUNK, CH), jnp.int32),
            pltpu.VMEM((NCHUNK, CH), jnp.int32),
            pltpu.VMEM((CH, D), jnp.float32),
            pltpu.VMEM((CH, CW), jnp.float32),
            pltpu.VMEM((125, D), jnp.float32),
            pltpu.VMEM((TPN, CW), jnp.float32),
            pltpu.SemaphoreType.DMA,
        ],
    )(x, srcg, dstg, zrow, zcnt, ones)


ROWS_BLK = 2500


def _combine_body(p_ref, c_ref, x_ref, wlT_ref, wrT_ref, b_ref, o_ref):
    cnt = c_ref[0, :, 0:1] + c_ref[1, :, 0:1]
    recip = 1.0 / jnp.maximum(cnt, 1.0)
    mean = (p_ref[0] + p_ref[1]) * recip
    h = (jnp.dot(mean, wlT_ref[...], preferred_element_type=jnp.float32)
         + jnp.dot(x_ref[...], wrT_ref[...], preferred_element_type=jnp.float32)
         + b_ref[...])
    o_ref[...] = jnp.maximum(h, 0.0)


def _tc_combine(p, c, x, wlT, wrT, b):
    grid = (N // ROWS_BLK,)
    return pl.pallas_call(
        _combine_body,
        grid=grid,
        in_specs=[
            pl.BlockSpec((NC, ROWS_BLK, D), lambda i: (0, i, 0)),
            pl.BlockSpec((NC, ROWS_BLK, CW), lambda i: (0, i, 0)),
            pl.BlockSpec((ROWS_BLK, D), lambda i: (i, 0)),
            pl.BlockSpec((D, D), lambda i: (0, 0)),
            pl.BlockSpec((D, D), lambda i: (0, 0)),
            pl.BlockSpec((1, D), lambda i: (0, 0)),
        ],
        out_specs=pl.BlockSpec((ROWS_BLK, D), lambda i: (i, 0)),
        out_shape=jax.ShapeDtypeStruct((N, D), jnp.float32),
    )(p, c, x, wlT, wrT, b)


def kernel(x, edge_index, Wl1, Wr1, b1, Wl2, Wr2, b2):
    src = edge_index[0].astype(jnp.int32).reshape(NW, NCHUNK, CH)
    dst = edge_index[1].astype(jnp.int32).reshape(NW, NCHUNK, CH)
    zrow = jnp.zeros((125, D), jnp.float32)
    zcnt = jnp.zeros((TPN, CW), jnp.float32)
    ones = jnp.ones((CH, CW), jnp.float32)

    p1, c1 = _sc_aggregate(x, src, dst, zrow, zcnt, ones)
    h1 = _tc_combine(p1, c1, x, Wl1.T, Wr1.T, b1.reshape(1, D))
    p2, c2 = _sc_aggregate(h1, src, dst, zrow, zcnt, ones)
    h2 = _tc_combine(p2, c2, h1, Wl2.T, Wr2.T, b2.reshape(1, D))
    return h2


# speedup vs baseline: 5.3666x; 5.3666x over previous
import jax
import jax.numpy as jnp
from jax import lax
from jax.experimental import pallas as pl
from jax.experimental.pallas import tpu as pltpu
from jax.experimental.pallas import tpu_sc as plsc

N = 10000
E = 320000
D = 128

NC = 2
NS = 16
HD = D // NC
EPT = E // NS
CH = 80
NCHUNK = EPT // CH
HCHUNK = NCHUNK // 2
NP = 10240
TPN = NP // NS
SCH = 128
CW = 16


def _agg_kernel_body(xh_hbm, srcg_hbm, dstg_hbm, zrow_hbm, zcnt_hbm, ones_hbm,
                     outp_hbm, outc_hbm,
                     src_v, dst_v, rows_v, ones_v, stage_v, cstage_v,
                     agg_sh, cnt_sh, sem):
    cid = lax.axis_index("c")
    sid = lax.axis_index("s")
    base = pl.multiple_of(sid * TPN, 8)

    pltpu.sync_copy(srcg_hbm.at[sid], src_v)
    pltpu.sync_copy(dstg_hbm.at[sid], dst_v)
    pltpu.sync_copy(ones_hbm, ones_v)
    pltpu.sync_copy(zrow_hbm, stage_v)
    pltpu.sync_copy(zcnt_hbm, cstage_v)

    for r in range(TPN // SCH):
        pltpu.sync_copy(stage_v, agg_sh.at[pl.ds(base + r * SCH, SCH)])
    pltpu.sync_copy(cstage_v, cnt_sh.at[pl.ds(base, TPN)])
    plsc.subcore_barrier()

    @pl.loop(0, NCHUNK)
    def _(j):
        pltpu.async_copy(xh_hbm.at[cid].at[src_v.at[j]], rows_v, sem).wait()
        pltpu.sync_copy(rows_v, agg_sh.at[dst_v.at[j]], add=True)

        do_cnt = lax.select(cid == 0, j < HCHUNK, j >= HCHUNK)

        @pl.when(do_cnt)
        def _():
            pltpu.sync_copy(ones_v, cnt_sh.at[dst_v.at[j]], add=True)

    plsc.subcore_barrier()

    for r in range(TPN // SCH):
        pltpu.sync_copy(agg_sh.at[pl.ds(base + r * SCH, SCH)], stage_v)
        pltpu.sync_copy(stage_v, outp_hbm.at[cid, pl.ds(base + r * SCH, SCH)])
    pltpu.sync_copy(cnt_sh.at[pl.ds(base, TPN)], cstage_v)
    pltpu.sync_copy(cstage_v, outc_hbm.at[cid, pl.ds(base, TPN)])


def _sc_aggregate(xh, srcg, dstg, zrow, zcnt, ones):
    mesh = plsc.VectorSubcoreMesh(core_axis_name="c", subcore_axis_name="s")
    return pl.kernel(
        _agg_kernel_body,
        out_type=[
            jax.ShapeDtypeStruct((NC, NP, HD), jnp.float32),
            jax.ShapeDtypeStruct((NC, NP, CW), jnp.float32),
        ],
        mesh=mesh,
        compiler_params=pltpu.CompilerParams(use_tc_tiling_on_sc=False),
        scratch_types=[
            pltpu.VMEM((NCHUNK, CH), jnp.int32),
            pltpu.VMEM((NCHUNK, CH), jnp.int32),
            pltpu.VMEM((CH, HD), jnp.float32),
            pltpu.VMEM((CH, CW), jnp.float32),
            pltpu.VMEM((SCH, HD), jnp.float32),
            pltpu.VMEM((TPN, CW), jnp.float32),
            pltpu.VMEM_SHARED((NP, HD), jnp.float32),
            pltpu.VMEM_SHARED((NP, CW), jnp.float32),
            pltpu.SemaphoreType.DMA,
        ],
    )(xh, srcg, dstg, zrow, zcnt, ones)


ROWS_BLK = 2000


def _combine_body(p_ref, c_ref, x_ref, wlT_ref, wrT_ref, b_ref, o_ref):
    cnt = c_ref[0, :, 0:1] + c_ref[1, :, 0:1]
    recip = 1.0 / jnp.maximum(cnt, 1.0)
    agg = jnp.concatenate([p_ref[0], p_ref[1]], axis=-1)
    mean = agg * recip
    h = (jnp.dot(mean, wlT_ref[...], preferred_element_type=jnp.float32)
         + jnp.dot(x_ref[...], wrT_ref[...], preferred_element_type=jnp.float32)
         + b_ref[...])
    o_ref[...] = jnp.maximum(h, 0.0)


def _tc_combine(p, c, x, wlT, wrT, b):
    grid = (N // ROWS_BLK,)
    return pl.pallas_call(
        _combine_body,
        grid=grid,
        in_specs=[
            pl.BlockSpec((NC, ROWS_BLK, HD), lambda i: (0, i, 0)),
            pl.BlockSpec((NC, ROWS_BLK, CW), lambda i: (0, i, 0)),
            pl.BlockSpec((ROWS_BLK, D), lambda i: (i, 0)),
            pl.BlockSpec((D, D), lambda i: (0, 0)),
            pl.BlockSpec((D, D), lambda i: (0, 0)),
            pl.BlockSpec((1, D), lambda i: (0, 0)),
        ],
        out_specs=pl.BlockSpec((ROWS_BLK, D), lambda i: (i, 0)),
        out_shape=jax.ShapeDtypeStruct((N, D), jnp.float32),
    )(p, c, x, wlT, wrT, b)


def _halves(x):
    return x.reshape(N, NC, HD).swapaxes(0, 1)


def kernel(x, edge_index, Wl1, Wr1, b1, Wl2, Wr2, b2):
    src = edge_index[0].astype(jnp.int32).reshape(NS, NCHUNK, CH)
    dst = edge_index[1].astype(jnp.int32).reshape(NS, NCHUNK, CH)
    zrow = jnp.zeros((SCH, HD), jnp.float32)
    zcnt = jnp.zeros((TPN, CW), jnp.float32)
    ones = jnp.ones((CH, CW), jnp.float32)

    p1, c1 = _sc_aggregate(_halves(x), src, dst, zrow, zcnt, ones)
    h1 = _tc_combine(p1, c1, x, Wl1.T, Wr1.T, b1.reshape(1, D))
    p2, c2 = _sc_aggregate(_halves(h1), src, dst, zrow, zcnt, ones)
    h2 = _tc_combine(p2, c2, h1, Wl2.T, Wr2.T, b2.reshape(1, D))
    return h2


# speedup vs baseline: 6.3817x; 1.1891x over previous
import jax
import jax.numpy as jnp
from jax import lax
from jax.experimental import pallas as pl
from jax.experimental.pallas import tpu as pltpu
from jax.experimental.pallas import tpu_sc as plsc

N = 10000
E = 320000
D = 128

NC = 2
NS = 16
HD = D // NC
EPT = E // NS
CH = 128
NCHUNK = 160
EPAD = NCHUNK * CH - EPT
HCHUNK = NCHUNK // 2
NBUF = 4
LA = 2
ROUNDS = NCHUNK // NBUF
CLAG = 8
NP = 10240
TPN = NP // NS
SCH = 128
CW = 16


def _make_agg_body(with_counts):
    def body(xh_hbm, srcg_hbm, dstg_hbm, zrow_hbm, zcnt_hbm, ones_hbm,
             outp_hbm, outc_hbm, *scratch):
        src_v, dst_v = scratch[0], scratch[1]
        rows = scratch[2:2 + NBUF]
        ones_v, agg_sh, cnt_sh = scratch[2 + NBUF:5 + NBUF]
        gsem = scratch[5 + NBUF:5 + 2 * NBUF]
        ssem = scratch[5 + 2 * NBUF:5 + 3 * NBUF]
        csem = scratch[5 + 3 * NBUF]

        cid = lax.axis_index("c")
        sid = lax.axis_index("s")
        base = pl.multiple_of(sid * TPN, 8)
        xc_hbm = xh_hbm.at[cid]
        lo = cid * HCHUNK

        pltpu.sync_copy(srcg_hbm.at[sid], src_v)
        pltpu.sync_copy(dstg_hbm.at[sid], dst_v)
        if with_counts:
            pltpu.sync_copy(ones_hbm, ones_v)

        pltpu.sync_copy(zrow_hbm, agg_sh.at[pl.ds(base, TPN)])
        if with_counts:
            pltpu.sync_copy(zcnt_hbm, cnt_sh.at[pl.ds(base, TPN)])
        plsc.subcore_barrier()

        for b in range(LA):
            pltpu.async_copy(xc_hbm.at[src_v.at[b]], rows[b], gsem[b])

        @pl.loop(0, ROUNDS)
        def _(g):
            for b in range(NBUF):
                j = g * NBUF + b
                bb = (b + LA) % NBUF
                jj = j + LA

                @pl.when(jj >= NBUF)
                def _():
                    pltpu.make_async_copy(
                        rows[bb], agg_sh.at[dst_v.at[0]], ssem[bb]).wait()

                @pl.when(jj < NCHUNK)
                def _():
                    pltpu.async_copy(xc_hbm.at[src_v.at[jj]], rows[bb], gsem[bb])

                pltpu.make_async_copy(
                    xc_hbm.at[src_v.at[j]], rows[b], gsem[b]).wait()
                pltpu.async_copy(rows[b], agg_sh.at[dst_v.at[j]], ssem[b],
                                 add=True)

                if with_counts:
                    @pl.when((j >= lo) & (j < lo + HCHUNK))
                    def _():
                        @pl.when(j >= lo + CLAG)
                        def _():
                            pltpu.make_async_copy(
                                ones_v, cnt_sh.at[dst_v.at[0]], csem).wait()
                        pltpu.async_copy(ones_v, cnt_sh.at[dst_v.at[j]], csem,
                                         add=True)

        for k in range(NCHUNK - (NBUF - LA), NCHUNK):
            pltpu.make_async_copy(
                rows[k % NBUF], agg_sh.at[dst_v.at[0]], ssem[k % NBUF]).wait()
        if with_counts:
            @pl.loop(0, CLAG)
            def _(t):
                pltpu.make_async_copy(ones_v, cnt_sh.at[dst_v.at[0]], csem).wait()
        plsc.subcore_barrier()

        pltpu.sync_copy(agg_sh.at[pl.ds(base, TPN)],
                        outp_hbm.at[cid, pl.ds(base, TPN)])
        if with_counts:
            pltpu.sync_copy(cnt_sh.at[pl.ds(base, TPN)],
                            outc_hbm.at[cid, pl.ds(base, TPN)])

    return body


def _sc_aggregate(xh, srcg, dstg, zrow, zcnt, ones, with_counts):
    mesh = plsc.VectorSubcoreMesh(core_axis_name="c", subcore_axis_name="s")
    return pl.kernel(
        _make_agg_body(with_counts),
        out_type=[
            jax.ShapeDtypeStruct((NC, NP, HD), jnp.float32),
            jax.ShapeDtypeStruct((NC, NP, CW), jnp.float32),
        ],
        mesh=mesh,
        compiler_params=pltpu.CompilerParams(use_tc_tiling_on_sc=False),
        scratch_types=(
            [
                pltpu.VMEM((NCHUNK, CH), jnp.int32),
                pltpu.VMEM((NCHUNK, CH), jnp.int32),
            ]
            + [pltpu.VMEM((CH, HD), jnp.float32) for _ in range(NBUF)]
            + [
                pltpu.VMEM((CH, CW), jnp.float32),
                pltpu.VMEM_SHARED((NP, HD), jnp.float32),
                pltpu.VMEM_SHARED((NP, CW), jnp.float32),
            ]
            + [pltpu.SemaphoreType.DMA for _ in range(2 * NBUF + 1)]
        ),
    )(xh, srcg, dstg, zrow, zcnt, ones)


ROWS_BLK = 2000


def _combine(p_ref, c_ref, x, wlT_ref, wrT_ref, b_ref):
    cnt = c_ref[0, :, 0:1] + c_ref[1, :, 0:1]
    recip = 1.0 / jnp.maximum(cnt, 1.0)
    agg = jnp.concatenate([p_ref[0], p_ref[1]], axis=-1)
    mean = agg * recip
    h = (jnp.dot(mean, wlT_ref[...], preferred_element_type=jnp.float32)
         + jnp.dot(x, wrT_ref[...], preferred_element_type=jnp.float32)
         + b_ref[...])
    return jnp.maximum(h, 0.0)


def _combine1_body(p_ref, c_ref, x_ref, wlT_ref, wrT_ref, b_ref, o_ref):
    h = _combine(p_ref, c_ref, x_ref[...], wlT_ref, wrT_ref, b_ref)
    o_ref[0] = h[:, :HD]
    o_ref[1] = h[:, HD:]


def _combine2_body(p_ref, c_ref, xh_ref, wlT_ref, wrT_ref, b_ref, o_ref):
    x = jnp.concatenate([xh_ref[0], xh_ref[1]], axis=-1)
    o_ref[...] = _combine(p_ref, c_ref, x, wlT_ref, wrT_ref, b_ref)


def _tc_combine(p, c, x, wlT, wrT, b, first):
    grid = (N // ROWS_BLK,)
    x_spec = (pl.BlockSpec((ROWS_BLK, D), lambda i: (i, 0)) if first
              else pl.BlockSpec((NC, ROWS_BLK, HD), lambda i: (0, i, 0)))
    out_spec = (pl.BlockSpec((NC, ROWS_BLK, HD), lambda i: (0, i, 0)) if first
                else pl.BlockSpec((ROWS_BLK, D), lambda i: (i, 0)))
    out_shape = (jax.ShapeDtypeStruct((NC, N, HD), jnp.float32) if first
                 else jax.ShapeDtypeStruct((N, D), jnp.float32))
    return pl.pallas_call(
        _combine1_body if first else _combine2_body,
        grid=grid,
        in_specs=[
            pl.BlockSpec((NC, ROWS_BLK, HD), lambda i: (0, i, 0)),
            pl.BlockSpec((NC, ROWS_BLK, CW), lambda i: (0, i, 0)),
            x_spec,
            pl.BlockSpec((D, D), lambda i: (0, 0)),
            pl.BlockSpec((D, D), lambda i: (0, 0)),
            pl.BlockSpec((1, D), lambda i: (0, 0)),
        ],
        out_specs=out_spec,
        out_shape=out_shape,
    )(p, c, x, wlT, wrT, b)


def kernel(x, edge_index, Wl1, Wr1, b1, Wl2, Wr2, b2):
    src = edge_index[0].astype(jnp.int32).reshape(NS, EPT)
    dst = edge_index[1].astype(jnp.int32).reshape(NS, EPT)
    srcp = jnp.pad(src, ((0, 0), (0, EPAD))).reshape(NS, NCHUNK, CH)
    dstp = jnp.pad(dst, ((0, 0), (0, EPAD)),
                   constant_values=NP - 1).reshape(NS, NCHUNK, CH)
    zrow = jnp.zeros((TPN, HD), jnp.float32)
    zcnt = jnp.zeros((TPN, CW), jnp.float32)
    ones = jnp.ones((CH, CW), jnp.float32)
    xh = x.reshape(N, NC, HD).swapaxes(0, 1)

    p1, c1 = _sc_aggregate(xh, srcp, dstp, zrow, zcnt, ones, True)
    h1h = _tc_combine(p1, c1, x, Wl1.T, Wr1.T, b1.reshape(1, D), True)
    p2, _ = _sc_aggregate(h1h, srcp, dstp, zrow, zcnt, ones, False)
    h2 = _tc_combine(p2, c1, h1h, Wl2.T, Wr2.T, b2.reshape(1, D), False)
    return h2


# speedup vs baseline: 9.7878x; 1.5337x over previous
import jax
import jax.numpy as jnp
from jax import lax
from jax.experimental import pallas as pl
from jax.experimental.pallas import tpu as pltpu
from jax.experimental.pallas import tpu_sc as plsc

N = 10000
E = 320000
D = 128

NC = 2
NS = 16
HD = D // NC
EPT = E // NS
CH = 128
NCHUNK = 160
EPAD = NCHUNK * CH - EPT
HCHUNK = NCHUNK // 2
NBUF = 4
LA = 2
NI = 8
IA = 4
ROUNDS = NCHUNK // NI
CLAG = 3
TPX = N // NS
NP = 10240
TPN = NP // NS
SCH = 128
CW = 16


def _make_agg_body(with_counts):
    def body(xh_hbm, idxg_hbm, zrow_hbm, zcnt_hbm, ones_hbm,
             outp_hbm, outc_hbm, *scratch):
        rows = scratch[0:NBUF]
        iring = scratch[NBUF:NBUF + NI]
        ones_v, x_sh, agg_sh, cnt_sh = scratch[NBUF + NI:NBUF + NI + 4]
        k = NBUF + NI + 4
        gsem = scratch[k:k + NBUF]
        ssem = scratch[k + NBUF:k + 2 * NBUF]
        isem = scratch[k + 2 * NBUF:k + 2 * NBUF + NI]
        csem = scratch[k + 2 * NBUF + NI]

        cid = lax.axis_index("c")
        sid = lax.axis_index("s")
        base = pl.multiple_of(sid * TPN, 8)
        xbase = sid * TPX
        lo = cid * HCHUNK

        if with_counts:
            pltpu.sync_copy(ones_hbm, ones_v)

        pltpu.sync_copy(xh_hbm.at[cid, pl.ds(xbase, TPX)],
                        x_sh.at[pl.ds(xbase, TPX)])
        pltpu.sync_copy(zrow_hbm, agg_sh.at[pl.ds(base, TPN)])
        if with_counts:
            pltpu.sync_copy(zcnt_hbm, cnt_sh.at[pl.ds(base, TPN)])
        plsc.subcore_barrier()

        def iwait(e):
            pltpu.make_async_copy(idxg_hbm.at[sid, 0], iring[e], isem[e]).wait()

        def gwait(b):
            pltpu.make_async_copy(x_sh.at[iring[0].at[0]], rows[b],
                                  gsem[b]).wait()

        def swait(b):
            pltpu.make_async_copy(rows[b], agg_sh.at[iring[0].at[1]],
                                  ssem[b]).wait()

        def cwait():
            pltpu.make_async_copy(ones_v, cnt_sh.at[iring[0].at[1]],
                                  csem).wait()

        for c in range(IA):
            pltpu.async_copy(idxg_hbm.at[sid, c], iring[c], isem[c])
        for b in range(LA):
            iwait(b)
            pltpu.async_copy(x_sh.at[iring[b].at[0]], rows[b], gsem[b])

        @pl.loop(0, ROUNDS)
        def _(g):
            for u in range(NI):
                j = g * NI + u
                bb = (u + LA) % NBUF
                e2 = (u + LA) % NI
                e3 = (u + IA) % NI
                jj = j + LA
                jjj = j + IA

                @pl.when(jjj < NCHUNK)
                def _():
                    pltpu.async_copy(idxg_hbm.at[sid, jjj], iring[e3], isem[e3])

                @pl.when(jj >= NBUF)
                def _():
                    swait(bb)

                @pl.when(jj < NCHUNK)
                def _():
                    iwait(e2)
                    pltpu.async_copy(x_sh.at[iring[e2].at[0]], rows[bb],
                                     gsem[bb])

                gwait(u % NBUF)
                pltpu.async_copy(rows[u % NBUF], agg_sh.at[iring[u].at[1]],
                                 ssem[u % NBUF], add=True)

                if with_counts:
                    @pl.when((j >= lo) & (j < lo + HCHUNK))
                    def _():
                        @pl.when(j >= lo + CLAG)
                        def _():
                            cwait()
                        pltpu.async_copy(ones_v, cnt_sh.at[iring[u].at[1]],
                                         csem, add=True)

        for kk in range(NCHUNK - (NBUF - LA), NCHUNK):
            swait(kk % NBUF)
        if with_counts:
            @pl.loop(0, CLAG)
            def _(t):
                cwait()
        plsc.subcore_barrier()

        pltpu.sync_copy(agg_sh.at[pl.ds(base, TPN)],
                        outp_hbm.at[cid, pl.ds(base, TPN)])
        if with_counts:
            pltpu.sync_copy(cnt_sh.at[pl.ds(base, TPN)],
                            outc_hbm.at[cid, pl.ds(base, TPN)])

    return body


def _sc_aggregate(xh, idxg, zrow, zcnt, ones, with_counts):
    mesh = plsc.VectorSubcoreMesh(core_axis_name="c", subcore_axis_name="s")
    return pl.kernel(
        _make_agg_body(with_counts),
        out_type=[
            jax.ShapeDtypeStruct((NC, NP, HD), jnp.float32),
            jax.ShapeDtypeStruct((NC, NP, CW), jnp.float32),
        ],
        mesh=mesh,
        compiler_params=pltpu.CompilerParams(use_tc_tiling_on_sc=False),
        scratch_types=(
            [pltpu.VMEM((CH, HD), jnp.float32) for _ in range(NBUF)]
            + [pltpu.VMEM((2, CH), jnp.int32) for _ in range(NI)]
            + [
                pltpu.VMEM((CH, CW), jnp.float32),
                pltpu.VMEM_SHARED((N, HD), jnp.float32),
                pltpu.VMEM_SHARED((NP, HD), jnp.float32),
                pltpu.VMEM_SHARED((NP, CW), jnp.float32),
            ]
            + [pltpu.SemaphoreType.DMA for _ in range(2 * NBUF + NI + 1)]
        ),
    )(xh, idxg, zrow, zcnt, ones)


ROWS_BLK = 2000


def _combine(p_ref, c_ref, x, wlT_ref, wrT_ref, b_ref):
    cnt = c_ref[0, :, 0:1] + c_ref[1, :, 0:1]
    recip = 1.0 / jnp.maximum(cnt, 1.0)
    agg = jnp.concatenate([p_ref[0], p_ref[1]], axis=-1)
    mean = agg * recip
    h = (jnp.dot(mean, wlT_ref[...], preferred_element_type=jnp.float32)
         + jnp.dot(x, wrT_ref[...], preferred_element_type=jnp.float32)
         + b_ref[...])
    return jnp.maximum(h, 0.0)


def _combine1_body(p_ref, c_ref, x_ref, wlT_ref, wrT_ref, b_ref, o_ref):
    h = _combine(p_ref, c_ref, x_ref[...], wlT_ref, wrT_ref, b_ref)
    o_ref[0] = h[:, :HD]
    o_ref[1] = h[:, HD:]


def _combine2_body(p_ref, c_ref, xh_ref, wlT_ref, wrT_ref, b_ref, o_ref):
    x = jnp.concatenate([xh_ref[0], xh_ref[1]], axis=-1)
    o_ref[...] = _combine(p_ref, c_ref, x, wlT_ref, wrT_ref, b_ref)


def _tc_combine(p, c, x, wlT, wrT, b, first):
    grid = (N // ROWS_BLK,)
    x_spec = (pl.BlockSpec((ROWS_BLK, D), lambda i: (i, 0)) if first
              else pl.BlockSpec((NC, ROWS_BLK, HD), lambda i: (0, i, 0)))
    out_spec = (pl.BlockSpec((NC, ROWS_BLK, HD), lambda i: (0, i, 0)) if first
                else pl.BlockSpec((ROWS_BLK, D), lambda i: (i, 0)))
    out_shape = (jax.ShapeDtypeStruct((NC, N, HD), jnp.float32) if first
                 else jax.ShapeDtypeStruct((N, D), jnp.float32))
    return pl.pallas_call(
        _combine1_body if first else _combine2_body,
        grid=grid,
        in_specs=[
            pl.BlockSpec((NC, ROWS_BLK, HD), lambda i: (0, i, 0)),
            pl.BlockSpec((NC, ROWS_BLK, CW), lambda i: (0, i, 0)),
            x_spec,
            pl.BlockSpec((D, D), lambda i: (0, 0)),
            pl.BlockSpec((D, D), lambda i: (0, 0)),
            pl.BlockSpec((1, D), lambda i: (0, 0)),
        ],
        out_specs=out_spec,
        out_shape=out_shape,
    )(p, c, x, wlT, wrT, b)


def kernel(x, edge_index, Wl1, Wr1, b1, Wl2, Wr2, b2):
    src = edge_index[0].astype(jnp.int32).reshape(NS, EPT)
    dst = edge_index[1].astype(jnp.int32).reshape(NS, EPT)
    srcp = jnp.pad(src, ((0, 0), (0, EPAD))).reshape(NS, NCHUNK, CH)
    dstp = jnp.pad(dst, ((0, 0), (0, EPAD)),
                   constant_values=NP - 1).reshape(NS, NCHUNK, CH)
    idxg = jnp.stack([srcp, dstp], axis=2)
    zrow = jnp.zeros((TPN, HD), jnp.float32)
    zcnt = jnp.zeros((TPN, CW), jnp.float32)
    ones = jnp.ones((CH, CW), jnp.float32)
    xh = x.reshape(N, NC, HD).swapaxes(0, 1)

    p1, c1 = _sc_aggregate(xh, idxg, zrow, zcnt, ones, True)
    h1h = _tc_combine(p1, c1, x, Wl1.T, Wr1.T, b1.reshape(1, D), True)
    p2, _ = _sc_aggregate(h1h, idxg, zrow, zcnt, ones, False)
    h2 = _tc_combine(p2, c1, h1h, Wl2.T, Wr2.T, b2.reshape(1, D), False)
    return h2


# speedup vs baseline: 10.8415x; 1.1076x over previous
import jax
import jax.numpy as jnp
from jax import lax
from jax.experimental import pallas as pl
from jax.experimental.pallas import tpu as pltpu
from jax.experimental.pallas import tpu_sc as plsc

N = 10000
E = 320000
D = 128

NC = 2
NS = 16
HD = D // NC
EPT = E // NS
CH = 128
NCHUNK = 160
EPAD = NCHUNK * CH - EPT
HCHUNK = NCHUNK // 2
NBUF = 4
LA = 2
NI = 8
IA = 4
ROUNDS = NCHUNK // NI
CLAG = 3
TPX = N // NS
NP = 10240
TPN = NP // NS
SCH = 128
CW = 16


def _make_agg_body(with_counts):
    def body(x_hbm, srcg_hbm, dstg_hbm, zrow_hbm, zcnt_hbm, ones_hbm,
             outp_hbm, outc_hbm, *scratch):
        rows = scratch[0:NBUF]
        iring = scratch[NBUF:NBUF + NI]
        ones_v, x_sh, agg_sh, cnt_sh = scratch[NBUF + NI:NBUF + NI + 4]
        k = NBUF + NI + 4
        gsem = scratch[k:k + NBUF]
        ssem = scratch[k + NBUF:k + 2 * NBUF]
        isem = scratch[k + 2 * NBUF:k + 2 * NBUF + NI]
        csem = scratch[k + 2 * NBUF + NI]

        cid = lax.axis_index("c")
        sid = lax.axis_index("s")
        base = pl.multiple_of(sid * TPN, 8)
        xbase = sid * TPX
        lo = cid * HCHUNK

        if with_counts:
            pltpu.sync_copy(ones_hbm, ones_v)

        @pl.when(cid == 0)
        def _():
            pltpu.sync_copy(x_hbm.at[pl.ds(xbase, TPX), pl.ds(0, HD)],
                            x_sh.at[pl.ds(xbase, TPX)])

        @pl.when(cid == 1)
        def _():
            pltpu.sync_copy(x_hbm.at[pl.ds(xbase, TPX), pl.ds(HD, HD)],
                            x_sh.at[pl.ds(xbase, TPX)])
        pltpu.sync_copy(zrow_hbm, agg_sh.at[pl.ds(base, TPN)])
        if with_counts:
            pltpu.sync_copy(zcnt_hbm, cnt_sh.at[pl.ds(base, TPN)])
        plsc.subcore_barrier()

        def iload(e, c):
            pltpu.async_copy(srcg_hbm.at[sid, c], iring[e].at[0], isem[e])
            pltpu.async_copy(dstg_hbm.at[sid, c], iring[e].at[1], isem[e])

        def iwait(e):
            pltpu.make_async_copy(srcg_hbm.at[sid, 0], iring[e].at[0],
                                  isem[e]).wait()
            pltpu.make_async_copy(dstg_hbm.at[sid, 0], iring[e].at[1],
                                  isem[e]).wait()

        def gwait(b):
            pltpu.make_async_copy(x_sh.at[iring[0].at[0]], rows[b],
                                  gsem[b]).wait()

        def swait(b):
            pltpu.make_async_copy(rows[b], agg_sh.at[iring[0].at[1]],
                                  ssem[b]).wait()

        def cwait():
            pltpu.make_async_copy(ones_v, cnt_sh.at[iring[0].at[1]],
                                  csem).wait()

        for c in range(IA):
            iload(c, c)
        for b in range(LA):
            iwait(b)
            pltpu.async_copy(x_sh.at[iring[b].at[0]], rows[b], gsem[b])

        @pl.loop(0, ROUNDS)
        def _(g):
            for u in range(NI):
                j = g * NI + u
                bb = (u + LA) % NBUF
                e2 = (u + LA) % NI
                e3 = (u + IA) % NI
                jj = j + LA
                jjj = j + IA

                @pl.when(jjj < NCHUNK)
                def _():
                    iload(e3, jjj)

                @pl.when(jj >= NBUF)
                def _():
                    swait(bb)

                @pl.when(jj < NCHUNK)
                def _():
                    iwait(e2)
                    pltpu.async_copy(x_sh.at[iring[e2].at[0]], rows[bb],
                                     gsem[bb])

                gwait(u % NBUF)
                pltpu.async_copy(rows[u % NBUF], agg_sh.at[iring[u].at[1]],
                                 ssem[u % NBUF], add=True)

                if with_counts:
                    @pl.when((j >= lo) & (j < lo + HCHUNK))
                    def _():
                        @pl.when(j >= lo + CLAG)
                        def _():
                            cwait()
                        pltpu.async_copy(ones_v, cnt_sh.at[iring[u].at[1]],
                                         csem, add=True)

        for kk in range(NCHUNK - (NBUF - LA), NCHUNK):
            swait(kk % NBUF)
        if with_counts:
            @pl.loop(0, CLAG)
            def _(t):
                cwait()
        plsc.subcore_barrier()

        pltpu.sync_copy(agg_sh.at[pl.ds(base, TPN)],
                        outp_hbm.at[cid, pl.ds(base, TPN)])
        if with_counts:
            pltpu.sync_copy(cnt_sh.at[pl.ds(base, TPN)],
                            outc_hbm.at[cid, pl.ds(base, TPN)])

    return body


def _sc_aggregate(x, srcg, dstg, zrow, zcnt, ones, with_counts):
    mesh = plsc.VectorSubcoreMesh(core_axis_name="c", subcore_axis_name="s")
    return pl.kernel(
        _make_agg_body(with_counts),
        out_type=[
            jax.ShapeDtypeStruct((NC, NP, HD), jnp.float32),
            jax.ShapeDtypeStruct((NC, NP, CW), jnp.float32),
        ],
        mesh=mesh,
        compiler_params=pltpu.CompilerParams(use_tc_tiling_on_sc=False),
        scratch_types=(
            [pltpu.VMEM((CH, HD), jnp.float32) for _ in range(NBUF)]
            + [pltpu.VMEM((2, CH), jnp.int32) for _ in range(NI)]
            + [
                pltpu.VMEM((CH, CW), jnp.float32),
                pltpu.VMEM_SHARED((N, HD), jnp.float32),
                pltpu.VMEM_SHARED((NP, HD), jnp.float32),
                pltpu.VMEM_SHARED((NP, CW), jnp.float32),
            ]
            + [pltpu.SemaphoreType.DMA for _ in range(2 * NBUF + NI + 1)]
        ),
    )(x, srcg, dstg, zrow, zcnt, ones)


ROWS_BLK = 2000


def _combine_body(p_ref, c_ref, x_ref, wlT_ref, wrT_ref, b_ref, o_ref):
    cnt = c_ref[0, :, 0:1] + c_ref[1, :, 0:1]
    recip = 1.0 / jnp.maximum(cnt, 1.0)
    agg = jnp.concatenate([p_ref[0], p_ref[1]], axis=-1)
    mean = agg * recip
    h = (jnp.dot(mean, wlT_ref[...], preferred_element_type=jnp.float32)
         + jnp.dot(x_ref[...], wrT_ref[...], preferred_element_type=jnp.float32)
         + b_ref[...])
    o_ref[...] = jnp.maximum(h, 0.0)


def _tc_combine(p, c, x, wlT, wrT, b):
    grid = (N // ROWS_BLK,)
    return pl.pallas_call(
        _combine_body,
        grid=grid,
        in_specs=[
            pl.BlockSpec((NC, ROWS_BLK, HD), lambda i: (0, i, 0)),
            pl.BlockSpec((NC, ROWS_BLK, CW), lambda i: (0, i, 0)),
            pl.BlockSpec((ROWS_BLK, D), lambda i: (i, 0)),
            pl.BlockSpec((D, D), lambda i: (0, 0)),
            pl.BlockSpec((D, D), lambda i: (0, 0)),
            pl.BlockSpec((1, D), lambda i: (0, 0)),
        ],
        out_specs=pl.BlockSpec((ROWS_BLK, D), lambda i: (i, 0)),
        out_shape=jax.ShapeDtypeStruct((N, D), jnp.float32),
    )(p, c, x, wlT, wrT, b)


def kernel(x, edge_index, Wl1, Wr1, b1, Wl2, Wr2, b2):
    src = edge_index[0].astype(jnp.int32).reshape(NS, EPT)
    dst = edge_index[1].astype(jnp.int32).reshape(NS, EPT)
    srcp = jnp.pad(src, ((0, 0), (0, EPAD))).reshape(NS, NCHUNK, CH)
    dstp = jnp.pad(dst, ((0, 0), (0, EPAD)),
                   constant_values=NP - 1).reshape(NS, NCHUNK, CH)
    zrow = jnp.zeros((TPN, HD), jnp.float32)
    zcnt = jnp.zeros((TPN, CW), jnp.float32)
    ones = jnp.ones((CH, CW), jnp.float32)

    p1, c1 = _sc_aggregate(x, srcp, dstp, zrow, zcnt, ones, True)
    h1 = _tc_combine(p1, c1, x, Wl1.T, Wr1.T, b1.reshape(1, D))
    p2, _ = _sc_aggregate(h1, srcp, dstp, zrow, zcnt, ones, False)
    h2 = _tc_combine(p2, c1, h1, Wl2.T, Wr2.T, b2.reshape(1, D))
    return h2


# speedup vs baseline: 11.6220x; 1.0720x over previous
import jax
import jax.numpy as jnp
from jax import lax
from jax.experimental import pallas as pl
from jax.experimental.pallas import tpu as pltpu
from jax.experimental.pallas import tpu_sc as plsc

N = 10000
E = 320000
D = 128

NC = 2
NS = 16
HD = D // NC
EPT = E // NS
CH = 128
NCHUNK = 160
EPAD = NCHUNK * CH - EPT
HCHUNK = NCHUNK // 2
NBUF = 4
LA = 2
NI = 8
IA = 4
ROUNDS = NCHUNK // NI
CLAG = 3
TPX = N // NS
NP = 10240
TPN = NP // NS
SCH = 128
CW = 16


def _make_agg_body(with_counts):
    def body(x_hbm, srcg_hbm, dstg_hbm, zrow_hbm, zcnt_hbm, ones_hbm,
             outp_hbm, outc_hbm, *scratch):
        rows = scratch[0:NBUF]
        iring = scratch[NBUF:NBUF + NI]
        ones_v, x_sh, agg_sh, cnt_sh = scratch[NBUF + NI:NBUF + NI + 4]
        k = NBUF + NI + 4
        gsem = scratch[k:k + NBUF]
        ssem = scratch[k + NBUF:k + 2 * NBUF]
        isem = scratch[k + 2 * NBUF:k + 2 * NBUF + NI]
        csem = scratch[k + 2 * NBUF + NI]

        cid = lax.axis_index("c")
        sid = lax.axis_index("s")
        base = pl.multiple_of(sid * TPN, 8)
        xbase = sid * TPX
        lo = cid * HCHUNK

        if with_counts:
            pltpu.sync_copy(ones_hbm, ones_v)

        @pl.when(cid == 0)
        def _():
            pltpu.sync_copy(x_hbm.at[pl.ds(xbase, TPX), pl.ds(0, HD)],
                            x_sh.at[pl.ds(xbase, TPX)])

        @pl.when(cid == 1)
        def _():
            pltpu.sync_copy(x_hbm.at[pl.ds(xbase, TPX), pl.ds(HD, HD)],
                            x_sh.at[pl.ds(xbase, TPX)])
        pltpu.sync_copy(zrow_hbm, agg_sh.at[pl.ds(base, TPN)])
        if with_counts:
            pltpu.sync_copy(zcnt_hbm, cnt_sh.at[pl.ds(base, TPN)])
        plsc.subcore_barrier()

        def iload(e, c):
            pltpu.async_copy(srcg_hbm.at[sid, c], iring[e].at[0], isem[e])
            pltpu.async_copy(dstg_hbm.at[sid, c], iring[e].at[1], isem[e])

        def iwait(e):
            pltpu.make_async_copy(srcg_hbm.at[sid, 0], iring[e].at[0],
                                  isem[e]).wait()
            pltpu.make_async_copy(dstg_hbm.at[sid, 0], iring[e].at[1],
                                  isem[e]).wait()

        def gwait(b):
            pltpu.make_async_copy(x_sh.at[iring[0].at[0]], rows[b],
                                  gsem[b]).wait()

        def swait(b):
            pltpu.make_async_copy(rows[b], agg_sh.at[iring[0].at[1]],
                                  ssem[b]).wait()

        def cwait():
            pltpu.make_async_copy(ones_v, cnt_sh.at[iring[0].at[1]],
                                  csem).wait()

        for c in range(IA):
            iload(c, c)
        for b in range(LA):
            iwait(b)
            pltpu.async_copy(x_sh.at[iring[b].at[0]], rows[b], gsem[b])

        @pl.loop(0, ROUNDS)
        def _(g):
            for u in range(NI):
                j = g * NI + u
                bb = (u + LA) % NBUF
                e2 = (u + LA) % NI
                e3 = (u + IA) % NI
                jj = j + LA
                jjj = j + IA

                @pl.when(jjj < NCHUNK)
                def _():
                    iload(e3, jjj)

                @pl.when(jj >= NBUF)
                def _():
                    swait(bb)

                @pl.when(jj < NCHUNK)
                def _():
                    iwait(e2)
                    pltpu.async_copy(x_sh.at[iring[e2].at[0]], rows[bb],
                                     gsem[bb])

                gwait(u % NBUF)
                pltpu.async_copy(rows[u % NBUF], agg_sh.at[iring[u].at[1]],
                                 ssem[u % NBUF], add=True)

                if with_counts:
                    @pl.when((j >= lo) & (j < lo + HCHUNK))
                    def _():
                        @pl.when(j >= lo + CLAG)
                        def _():
                            cwait()
                        pltpu.async_copy(ones_v, cnt_sh.at[iring[u].at[1]],
                                         csem, add=True)

        for kk in range(NCHUNK - (NBUF - LA), NCHUNK):
            swait(kk % NBUF)
        if with_counts:
            @pl.loop(0, CLAG)
            def _(t):
                cwait()
        plsc.subcore_barrier()

        @pl.when(cid == 0)
        def _():
            pltpu.sync_copy(agg_sh.at[pl.ds(base, TPN)],
                            outp_hbm.at[pl.ds(base, TPN), pl.ds(0, HD)])
            if with_counts:
                pltpu.sync_copy(cnt_sh.at[pl.ds(base, TPN)],
                                outc_hbm.at[pl.ds(base, TPN), pl.ds(0, CW)])

        @pl.when(cid == 1)
        def _():
            pltpu.sync_copy(agg_sh.at[pl.ds(base, TPN)],
                            outp_hbm.at[pl.ds(base, TPN), pl.ds(HD, HD)])
            if with_counts:
                pltpu.sync_copy(cnt_sh.at[pl.ds(base, TPN)],
                                outc_hbm.at[pl.ds(base, TPN), pl.ds(CW, CW)])

    return body


def _sc_aggregate(x, srcg, dstg, zrow, zcnt, ones, with_counts):
    mesh = plsc.VectorSubcoreMesh(core_axis_name="c", subcore_axis_name="s")
    return pl.kernel(
        _make_agg_body(with_counts),
        out_type=[
            jax.ShapeDtypeStruct((NP, D), jnp.float32),
            jax.ShapeDtypeStruct((NP, D), jnp.float32),
        ],
        mesh=mesh,
        compiler_params=pltpu.CompilerParams(use_tc_tiling_on_sc=False),
        scratch_types=(
            [pltpu.VMEM((CH, HD), jnp.float32) for _ in range(NBUF)]
            + [pltpu.VMEM((2, CH), jnp.int32) for _ in range(NI)]
            + [
                pltpu.VMEM((CH, CW), jnp.float32),
                pltpu.VMEM_SHARED((N, HD), jnp.float32),
                pltpu.VMEM_SHARED((NP, HD), jnp.float32),
                pltpu.VMEM_SHARED((NP, CW), jnp.float32),
            ]
            + [pltpu.SemaphoreType.DMA for _ in range(2 * NBUF + NI + 1)]
        ),
    )(x, srcg, dstg, zrow, zcnt, ones)


ROWS_BLK = 2000


def _combine_body(p_ref, c_ref, x_ref, wlT_ref, wrT_ref, b_ref, o_ref):
    cnt = c_ref[:, 0:1] + c_ref[:, CW:CW + 1]
    recip = 1.0 / jnp.maximum(cnt, 1.0)
    mean = p_ref[...] * recip
    h = (jnp.dot(mean, wlT_ref[...], preferred_element_type=jnp.float32)
         + jnp.dot(x_ref[...], wrT_ref[...], preferred_element_type=jnp.float32)
         + b_ref[...])
    o_ref[...] = jnp.maximum(h, 0.0)


def _tc_combine(p, c, x, wlT, wrT, b):
    grid = (N // ROWS_BLK,)
    return pl.pallas_call(
        _combine_body,
        grid=grid,
        in_specs=[
            pl.BlockSpec((ROWS_BLK, D), lambda i: (i, 0)),
            pl.BlockSpec((ROWS_BLK, D), lambda i: (i, 0)),
            pl.BlockSpec((ROWS_BLK, D), lambda i: (i, 0)),
            pl.BlockSpec((D, D), lambda i: (0, 0)),
            pl.BlockSpec((D, D), lambda i: (0, 0)),
            pl.BlockSpec((1, D), lambda i: (0, 0)),
        ],
        out_specs=pl.BlockSpec((ROWS_BLK, D), lambda i: (i, 0)),
        out_shape=jax.ShapeDtypeStruct((N, D), jnp.float32),
    )(p, c, x, wlT, wrT, b)


def kernel(x, edge_index, Wl1, Wr1, b1, Wl2, Wr2, b2):
    src = edge_index[0].astype(jnp.int32).reshape(NS, EPT)
    dst = edge_index[1].astype(jnp.int32).reshape(NS, EPT)
    srcp = jnp.pad(src, ((0, 0), (0, EPAD))).reshape(NS, NCHUNK, CH)
    dstp = jnp.pad(dst, ((0, 0), (0, EPAD)),
                   constant_values=NP - 1).reshape(NS, NCHUNK, CH)
    zrow = jnp.zeros((TPN, HD), jnp.float32)
    zcnt = jnp.zeros((TPN, CW), jnp.float32)
    ones = jnp.ones((CH, CW), jnp.float32)

    p1, c1 = _sc_aggregate(x, srcp, dstp, zrow, zcnt, ones, True)
    h1 = _tc_combine(p1, c1, x, Wl1.T, Wr1.T, b1.reshape(1, D))
    p2, _ = _sc_aggregate(h1, srcp, dstp, zrow, zcnt, ones, False)
    h2 = _tc_combine(p2, c1, h1, Wl2.T, Wr2.T, b2.reshape(1, D))
    return h2
